# initial kernel scaffold (unmeasured)
import jax
import jax.numpy as jnp
from jax import lax
from jax.experimental import pallas as pl
from jax.experimental.pallas import tpu as pltpu

N_Y = 4


def kernel(x, W):
    T, D = x.shape
    _, V_sh = W.shape
    V = N_Y * V_sh

    x = x.astype(jnp.bfloat16)
    W = W.astype(jnp.bfloat16)

    def body(x_ref, w_ref, out_ref, logits_ref, send_sems, recv_sems):
        my_x = lax.axis_index("x")
        my_y = lax.axis_index("y")
        my_z = lax.axis_index("z")
        left = (my_y - 1) % N_Y
        right = (my_y + 1) % N_Y

        logits = jnp.dot(
            x_ref[...], w_ref[...], preferred_element_type=jnp.float32
        )
        logits_ref[pl.ds(my_y, 1)] = logits.astype(jnp.bfloat16)[None]

        barrier_sem = pltpu.get_barrier_semaphore()
        for nbr in (left, right):
            pl.semaphore_signal(
                barrier_sem,
                inc=1,
                device_id=(my_x, nbr, my_z),
                device_id_type=pl.DeviceIdType.MESH,
            )
        pl.semaphore_wait(barrier_sem, 2)

        for h in range(N_Y - 1):
            origin = (my_y - h) % N_Y
            rdma = pltpu.make_async_remote_copy(
                src_ref=logits_ref.at[origin],
                dst_ref=logits_ref.at[origin],
                send_sem=send_sems.at[h],
                recv_sem=recv_sems.at[h],
                device_id=(my_x, right, my_z),
                device_id_type=pl.DeviceIdType.MESH,
            )
            rdma.start()
            rdma.wait()

        m = jnp.full((T, 1), -jnp.inf, dtype=jnp.float32)
        for c in range(N_Y):
            m = jnp.maximum(
                m,
                jnp.max(
                    logits_ref[c].astype(jnp.float32), axis=1, keepdims=True
                ),
            )
        s = jnp.zeros((T, 1), dtype=jnp.float32)
        for c in range(N_Y):
            e = jnp.exp(logits_ref[c].astype(jnp.float32) - m)
            s = s + jnp.sum(e, axis=1, keepdims=True)
            out_ref[:, c * V_sh : (c + 1) * V_sh] = e
        r = 1.0 / s
        for c in range(N_Y):
            out_ref[:, c * V_sh : (c + 1) * V_sh] = (
                out_ref[:, c * V_sh : (c + 1) * V_sh] * r
            )

    return pl.pallas_call(
        body,
        out_shape=jax.ShapeDtypeStruct((T, V), jnp.float32),
        in_specs=[
            pl.BlockSpec(memory_space=pltpu.VMEM),
            pl.BlockSpec(memory_space=pltpu.VMEM),
        ],
        out_specs=pl.BlockSpec(memory_space=pltpu.VMEM),
        scratch_shapes=[
            pltpu.VMEM((N_Y, T, V_sh), jnp.bfloat16),
            pltpu.SemaphoreType.DMA((N_Y - 1,)),
            pltpu.SemaphoreType.DMA((N_Y - 1,)),
        ],
        compiler_params=pltpu.CompilerParams(collective_id=0),
    )(x, W)


# baseline (device time: 402318 ns/iter reference)
import jax
import jax.numpy as jnp
from jax import lax
from jax.experimental import pallas as pl
from jax.experimental.pallas import tpu as pltpu

N_Y = 4
TILE = 1024


def kernel(x, W):
    T, D = x.shape
    _, V_sh = W.shape
    V = N_Y * V_sh
    n_sub = V_sh // TILE
    n_tiles = V // TILE

    x = x.astype(jnp.bfloat16)
    W = W.astype(jnp.bfloat16)

    def body(x_ref, w_hbm, out_ref, logits_ref, wbuf_ref, stage_ref,
             send_sems, recv_sems, w_sems, out_sems):
        my_x = lax.axis_index("x")
        my_y = lax.axis_index("y")
        my_z = lax.axis_index("z")
        left = (my_y - 1) % N_Y
        right = (my_y + 1) % N_Y

        def w_copy(i):
            slot = lax.rem(i, 2)
            return pltpu.make_async_copy(
                w_hbm.at[:, pl.ds(i * TILE, TILE)],
                wbuf_ref.at[slot],
                w_sems.at[slot],
            )

        w_copy(0).start()
        w_copy(1).start()

        def gemm_step(i, carry):
            slot = lax.rem(i, 2)
            w_copy(i).wait()
            t = jnp.dot(
                x_ref[...], wbuf_ref[slot],
                preferred_element_type=jnp.float32,
            )
            logits_ref[pl.ds(my_y * n_sub + i, 1)] = t.astype(jnp.bfloat16)[
                None
            ]

            @pl.when(i + 2 < n_sub)
            def _():
                w_copy(i + 2).start()

            return carry

        lax.fori_loop(0, n_sub, gemm_step, 0)

        barrier_sem = pltpu.get_barrier_semaphore()
        for nbr in (left, right):
            pl.semaphore_signal(
                barrier_sem,
                inc=1,
                device_id=(my_x, nbr, my_z),
                device_id_type=pl.DeviceIdType.MESH,
            )
        pl.semaphore_wait(barrier_sem, 2)

        for h in range(N_Y - 1):
            origin = (my_y - h) % N_Y
            rdma = pltpu.make_async_remote_copy(
                src_ref=logits_ref.at[pl.ds(origin * n_sub, n_sub)],
                dst_ref=logits_ref.at[pl.ds(origin * n_sub, n_sub)],
                send_sem=send_sems.at[h],
                recv_sem=recv_sems.at[h],
                device_id=(my_x, right, my_z),
                device_id_type=pl.DeviceIdType.MESH,
            )
            rdma.start()
            rdma.wait()

        def max_step(t, m):
            blk = logits_ref[t].astype(jnp.float32)
            return jnp.maximum(m, jnp.max(blk, axis=1, keepdims=True))

        m = lax.fori_loop(
            0, n_tiles, max_step,
            jnp.full((T, 1), -jnp.inf, dtype=jnp.float32),
        )

        def exp_step(t, s):
            e = jnp.exp(logits_ref[t].astype(jnp.float32) - m)
            logits_ref[pl.ds(t, 1)] = e.astype(jnp.bfloat16)[None]
            return s + jnp.sum(e, axis=1, keepdims=True)

        s = lax.fori_loop(
            0, n_tiles, exp_step, jnp.zeros((T, 1), dtype=jnp.float32)
        )
        r = 1.0 / s

        def out_copy(t):
            slot = lax.rem(t, 2)
            return pltpu.make_async_copy(
                stage_ref.at[slot],
                out_ref.at[:, pl.ds(t * TILE, TILE)],
                out_sems.at[slot],
            )

        def stage_tile(t):
            slot = lax.rem(t, 2)
            stage_ref[pl.ds(slot, 1)] = (
                logits_ref[t].astype(jnp.float32) * r
            )[None]
            out_copy(t).start()

        stage_tile(0)
        stage_tile(1)

        def out_step(t, carry):
            out_copy(t - 2).wait()
            stage_tile(t)
            return carry

        lax.fori_loop(2, n_tiles, out_step, 0)
        out_copy(n_tiles - 2).wait()
        out_copy(n_tiles - 1).wait()

    return pl.pallas_call(
        body,
        out_shape=jax.ShapeDtypeStruct((T, V), jnp.float32),
        in_specs=[
            pl.BlockSpec(memory_space=pltpu.VMEM),
            pl.BlockSpec(memory_space=pl.ANY),
        ],
        out_specs=pl.BlockSpec(memory_space=pl.ANY),
        scratch_shapes=[
            pltpu.VMEM((n_tiles, T, TILE), jnp.bfloat16),
            pltpu.VMEM((2, D, TILE), jnp.bfloat16),
            pltpu.VMEM((2, T, TILE), jnp.float32),
            pltpu.SemaphoreType.DMA((N_Y - 1,)),
            pltpu.SemaphoreType.DMA((N_Y - 1,)),
            pltpu.SemaphoreType.DMA((2,)),
            pltpu.SemaphoreType.DMA((2,)),
        ],
        compiler_params=pltpu.CompilerParams(
            collective_id=0, vmem_limit_bytes=64 * 1024 * 1024
        ),
    )(x, W)


# device time: 387442 ns/iter; 1.0384x vs baseline; 1.0384x over previous
import jax
import jax.numpy as jnp
from jax import lax
from jax.experimental import pallas as pl
from jax.experimental.pallas import tpu as pltpu

N_Y = 4
TILE = 1024


def kernel(x, W):
    T, D = x.shape
    _, V_sh = W.shape
    V = N_Y * V_sh
    n_sub = V_sh // TILE
    n_tiles = V // TILE

    x = x.astype(jnp.bfloat16)
    W = W.astype(jnp.bfloat16)

    def body(x_ref, w_hbm, out_ref, logits_ref, wbuf_ref, stage_ref,
             stats_ref, send_sems, recv_sems, st_send_sems, st_recv_sems,
             w_sems, out_sems):
        my_x = lax.axis_index("x")
        my_y = lax.axis_index("y")
        my_z = lax.axis_index("z")
        left = (my_y - 1) % N_Y
        right = (my_y + 1) % N_Y

        def w_copy(i):
            slot = lax.rem(i, 2)
            return pltpu.make_async_copy(
                w_hbm.at[:, pl.ds(i * TILE, TILE)],
                wbuf_ref.at[slot],
                w_sems.at[slot],
            )

        w_copy(0).start()
        w_copy(1).start()

        def gemm_step(i, m):
            slot = lax.rem(i, 2)
            w_copy(i).wait()
            v = jnp.dot(
                x_ref[...], wbuf_ref[slot],
                preferred_element_type=jnp.float32,
            )
            logits_ref[pl.ds(my_y * n_sub + i, 1)] = v.astype(jnp.bfloat16)[
                None
            ]
            m = jnp.maximum(m, jnp.max(v, axis=1, keepdims=True))

            @pl.when(i + 2 < n_sub)
            def _():
                w_copy(i + 2).start()

            return m

        m = lax.fori_loop(
            0, n_sub, gemm_step,
            jnp.full((T, 1), -jnp.inf, dtype=jnp.float32),
        )

        def exp_step(i, s):
            t = my_y * n_sub + i
            e = jnp.exp(logits_ref[t].astype(jnp.float32) - m)
            logits_ref[pl.ds(t, 1)] = e.astype(jnp.bfloat16)[None]
            return s + jnp.sum(e, axis=1, keepdims=True)

        s = lax.fori_loop(
            0, n_sub, exp_step, jnp.zeros((T, 1), dtype=jnp.float32)
        )
        stats_ref[pl.ds(my_y, 1)] = jnp.concatenate([m, s], axis=1)[None]

        barrier_sem = pltpu.get_barrier_semaphore()
        for nbr in (left, right):
            pl.semaphore_signal(
                barrier_sem,
                inc=1,
                device_id=(my_x, nbr, my_z),
                device_id_type=pl.DeviceIdType.MESH,
            )
        pl.semaphore_wait(barrier_sem, 2)

        def chunk_rdma(h):
            origin = (my_y - h) % N_Y
            return pltpu.make_async_remote_copy(
                src_ref=logits_ref.at[pl.ds(origin * n_sub, n_sub)],
                dst_ref=logits_ref.at[pl.ds(origin * n_sub, n_sub)],
                send_sem=send_sems.at[h],
                recv_sem=recv_sems.at[h],
                device_id=(my_x, right, my_z),
                device_id_type=pl.DeviceIdType.MESH,
            )

        chunk_rdma(0).start()

        for h in range(N_Y - 1):
            origin = (my_y - h) % N_Y
            st = pltpu.make_async_remote_copy(
                src_ref=stats_ref.at[pl.ds(origin, 1)],
                dst_ref=stats_ref.at[pl.ds(origin, 1)],
                send_sem=st_send_sems.at[h],
                recv_sem=st_recv_sems.at[h],
                device_id=(my_x, right, my_z),
                device_id_type=pl.DeviceIdType.MESH,
            )
            st.start()
            st.wait()

        ms = [stats_ref[c] for c in range(N_Y)]
        M = ms[0][:, 0:1]
        for c in range(1, N_Y):
            M = jnp.maximum(M, ms[c][:, 0:1])
        S = jnp.zeros((T, 1), dtype=jnp.float32)
        for c in range(N_Y):
            S = S + ms[c][:, 1:2] * jnp.exp(ms[c][:, 0:1] - M)
        rS = 1.0 / S

        def stream_chunk(c):
            st = stats_ref[c]
            factor = jnp.exp(st[:, 0:1] - M) * rS

            def out_copy(i):
                slot = lax.rem(i, 2)
                return pltpu.make_async_copy(
                    stage_ref.at[slot],
                    out_ref.at[:, pl.ds((c * n_sub + i) * TILE, TILE)],
                    out_sems.at[slot],
                )

            def stage_tile(i):
                slot = lax.rem(i, 2)
                e = logits_ref[c * n_sub + i].astype(jnp.float32)
                stage_ref[pl.ds(slot, 1)] = (e * factor)[None]
                out_copy(i).start()

            stage_tile(0)
            stage_tile(1)

            def step(i, carry):
                out_copy(i - 2).wait()
                stage_tile(i)
                return carry

            lax.fori_loop(2, n_sub, step, 0)
            out_copy(n_sub - 2).wait()
            out_copy(n_sub - 1).wait()

        stream_chunk(my_y)

        for h in range(N_Y - 1):
            rdma = chunk_rdma(h)
            rdma.wait()
            if h + 1 < N_Y - 1:
                chunk_rdma(h + 1).start()
            stream_chunk((my_y - h - 1) % N_Y)

    return pl.pallas_call(
        body,
        out_shape=jax.ShapeDtypeStruct((T, V), jnp.float32),
        in_specs=[
            pl.BlockSpec(memory_space=pltpu.VMEM),
            pl.BlockSpec(memory_space=pl.ANY),
        ],
        out_specs=pl.BlockSpec(memory_space=pl.ANY),
        scratch_shapes=[
            pltpu.VMEM((n_tiles, T, TILE), jnp.bfloat16),
            pltpu.VMEM((2, D, TILE), jnp.bfloat16),
            pltpu.VMEM((2, T, TILE), jnp.float32),
            pltpu.VMEM((N_Y, T, 2), jnp.float32),
            pltpu.SemaphoreType.DMA((N_Y - 1,)),
            pltpu.SemaphoreType.DMA((N_Y - 1,)),
            pltpu.SemaphoreType.DMA((N_Y - 1,)),
            pltpu.SemaphoreType.DMA((N_Y - 1,)),
            pltpu.SemaphoreType.DMA((2,)),
            pltpu.SemaphoreType.DMA((2,)),
        ],
        compiler_params=pltpu.CompilerParams(
            collective_id=0, vmem_limit_bytes=64 * 1024 * 1024
        ),
    )(x, W)
